# baseline (speedup 1.0000x reference)
import functools

import jax
import jax.numpy as jnp
from jax import lax
from jax.experimental import pallas as pl
from jax.experimental.pallas import tpu as pltpu


def _pick_tile(v):
    for tv in (1024, 512, 256, 128, 64, 32, 16, 8):
        if v % tv == 0:
            return tv
    return v


def _merge_scatter_kernel(sids_ref, order_ref, starts_ref,
                          w0_ref, w1_ref, w2_ref, m_in_ref, m_out_ref, s0_ref,
                          out_hbm, merged, sems, *, tv, nv):
    i = pl.program_id(0)
    slot = lax.rem(i, 3)

    def wait_rows(n, sl):
        def body8(_, c):
            pltpu.make_async_copy(
                merged.at[sl, pl.ds(0, 8)], out_hbm.at[pl.ds(0, 8)],
                sems.at[sl]).wait()
            return c
        lax.fori_loop(0, n >> 3, body8, 0)

        def body1(_, c):
            pltpu.make_async_copy(
                merged.at[sl, pl.ds(0, 1)], out_hbm.at[pl.ds(0, 1)],
                sems.at[sl]).wait()
            return c
        lax.fori_loop(0, n & 7, body1, 0)

    @pl.when(i >= 3)
    def _():
        wait_rows(starts_ref[i - 2] - starts_ref[i - 3], slot)

    m_col = jnp.transpose(m_out_ref[0], (1, 0))
    merged[slot] = (w0_ref[...] * s0_ref[0] + w1_ref[...] * m_in_ref[...]
                    + w2_ref[...] * m_col)

    lo = starts_ref[i]
    base = i * tv

    n = starts_ref[i + 1] - lo

    def start_row(idx):
        row = sids_ref[idx] - base
        tok = order_ref[idx]
        pltpu.make_async_copy(
            merged.at[slot, pl.ds(row, 1)], out_hbm.at[pl.ds(tok, 1)],
            sems.at[slot]).start()

    def issue4(c, carry):
        for u in range(4):
            start_row(lo + c * 4 + u)
        return carry
    lax.fori_loop(0, n >> 2, issue4, 0)

    def issue1(k, carry):
        start_row(lo + (n & ~3) + k)
        return carry
    lax.fori_loop(0, n & 3, issue1, 0)

    @pl.when(i == nv - 1)
    def _():
        for j in range(max(nv - 3, 0), nv):
            wait_rows(starts_ref[j + 1] - starts_ref[j], lax.rem(j, 3))


def kernel(input_ids, w0, w1, w2, scalar_mask, vec_in_mask, vec_out_mask):
    B, S = input_ids.shape
    V, H = w0.shape
    dtype = w0.dtype
    T = B * S

    ids = input_ids.reshape(T).astype(jnp.int32)
    sids, order = lax.sort([ids, lax.iota(jnp.int32, T)], num_keys=1)
    tv = _pick_tile(V)
    nv = V // tv
    bounds = jnp.arange(nv + 1, dtype=jnp.int32) * tv
    starts = jnp.sum(sids[None, :] < bounds[:, None], axis=1,
                     dtype=jnp.int32)

    m_in = jnp.asarray(vec_in_mask, dtype).reshape(1, H)
    m_out = jnp.asarray(vec_out_mask, dtype).reshape(nv, 1, tv)
    s0 = jnp.asarray(scalar_mask, dtype).reshape(1)

    grid_spec = pltpu.PrefetchScalarGridSpec(
        num_scalar_prefetch=3,
        grid=(nv,),
        in_specs=[
            pl.BlockSpec((tv, H), lambda i, *_: (i, 0)),
            pl.BlockSpec((tv, H), lambda i, *_: (i, 0)),
            pl.BlockSpec((tv, H), lambda i, *_: (i, 0)),
            pl.BlockSpec((1, H), lambda i, *_: (0, 0)),
            pl.BlockSpec((1, 1, tv), lambda i, *_: (i, 0, 0)),
            pl.BlockSpec(memory_space=pltpu.MemorySpace.SMEM),
        ],
        out_specs=pl.BlockSpec(memory_space=pl.ANY),
        scratch_shapes=[
            pltpu.VMEM((3, tv, H), dtype),
            pltpu.SemaphoreType.DMA((3,)),
        ],
    )
    out = pl.pallas_call(
        functools.partial(_merge_scatter_kernel, tv=tv, nv=nv),
        out_shape=jax.ShapeDtypeStruct((T, H), dtype),
        grid_spec=grid_spec,
        compiler_params=pltpu.CompilerParams(
            dimension_semantics=("arbitrary",),
            disable_bounds_checks=True),
        name="merge_scatter_stream",
    )(sids, order, starts, w0, w1, w2, m_in, m_out, s0)
    return out.reshape(B, S, H)
